# baseline (speedup 1.0000x reference)
import functools

import jax
import jax.numpy as jnp
from jax import lax
from jax.experimental import pallas as pl
from jax.experimental.pallas import tpu as pltpu
from jax.experimental.pallas import tpu_sc as plsc

B = 16384
D = 64
NC = 2
NS = 16
NW = NC * NS
CHUNK = B // NW

_TWO_PI = 6.283185307179586


def _fourier_tc(x_ref, f_ref, o_ref):
    x = x_ref[:]
    f = f_ref[:]
    ang = _TWO_PI * x[:, None] * f
    o_ref[:, : D // 2] = jnp.sin(ang)
    o_ref[:, D // 2 :] = jnp.cos(ang)


def _fourier_pallas(vals, freqs):
    n = vals.shape[0]
    blk = 2048
    return pl.pallas_call(
        _fourier_tc,
        grid=(n // blk,),
        in_specs=[
            pl.BlockSpec((blk,), lambda i: (i,)),
            pl.BlockSpec((1, D // 2), lambda i: (0, 0)),
        ],
        out_specs=pl.BlockSpec((blk, D), lambda i: (i, 0)),
        out_shape=jax.ShapeDtypeStruct((n, D), jnp.float32),
    )(vals, freqs.reshape(1, D // 2))


def _sc_body(time_emb, xt, dose_emb,
             r_idx, a_idx, c_idx, e_idx, w_idx, g_idx, m_idx,
             r_tab, a_tab, c_tab, e_tab, w_tab, g_tab, m_tab,
             out, *scratch):
    idxb = scratch[0:11]
    bufs = scratch[11:13]
    gsems = scratch[13:15]
    ssems = scratch[15:17]
    csem = scratch[17]
    isem = scratch[18]

    wid = lax.axis_index("s") * NC + lax.axis_index("c")
    base = wid * CHUNK

    cds = [
        pltpu.async_copy(time_emb.at[pl.ds(base, CHUNK)],
                         out.at[0, pl.ds(base, CHUNK)], csem),
        pltpu.async_copy(xt.at[pl.ds(base, CHUNK)],
                         out.at[1, pl.ds(base, CHUNK)], csem),
    ]
    for j in range(3):
        cds.append(pltpu.async_copy(dose_emb.at[pl.ds(j * B + base, CHUNK)],
                                    out.at[13 + j, pl.ds(base, CHUNK)], csem))

    jobs = [
        (2, r_idx, 0, r_tab),
        (3, a_idx, 0, a_tab),
        (4, c_idx, 0, c_tab),
        (5, e_idx, 0, e_tab),
        (6, w_idx, 0, w_tab),
        (7, g_idx, 0, g_tab),
        (8, g_idx, B, g_tab),
        (9, g_idx, 2 * B, g_tab),
        (10, m_idx, 0, m_tab),
        (11, m_idx, B, m_tab),
        (12, m_idx, 2 * B, m_tab),
    ]

    ids_ = []
    for j, (slot, idx_hbm, off, tab) in enumerate(jobs):
        src = idx_hbm.at[pl.ds(off + base, CHUNK)]
        ids_.append(pltpu.async_copy(src, idxb[j], isem))

    sds = [None, None]
    prev = None
    for j, (slot, idx_hbm, off, tab) in enumerate(jobs):
        b = j % 2
        if sds[b] is not None:
            sds[b].wait()
        ids_[j].wait()
        gd = pltpu.async_copy(tab.at[idxb[j]], bufs[b], gsems[b])
        if prev is not None:
            pgd, pslot, pb = prev
            pgd.wait()
            sds[pb] = pltpu.async_copy(bufs[pb],
                                       out.at[pslot, pl.ds(base, CHUNK)],
                                       ssems[pb])
        prev = (gd, slot, b)

    pgd, pslot, pb = prev
    pgd.wait()
    sds[pb] = pltpu.async_copy(bufs[pb], out.at[pslot, pl.ds(base, CHUNK)],
                               ssems[pb])
    for sd in sds:
        if sd is not None:
            sd.wait()
    for cd in cds:
        cd.wait()


@functools.cache
def _sc_assemble():
    return pl.kernel(
        _sc_body,
        out_type=jax.ShapeDtypeStruct((16, B, D), jnp.float32),
        mesh=plsc.VectorSubcoreMesh(core_axis_name="c", subcore_axis_name="s",
                                    num_cores=NC, num_subcores=NS),
        scratch_types=(
            [pltpu.VMEM((CHUNK,), jnp.int32) for _ in range(11)]
            + [pltpu.VMEM((CHUNK, D), jnp.float32) for _ in range(2)]
            + [pltpu.SemaphoreType.DMA] * 6
        ),
        compiler_params=pltpu.CompilerParams(use_tc_tiling_on_sc=False),
    )


def kernel(time, xt, routing_idx, assay_idx, cell_type_idx, experiment_idx,
           well_idx, gene_pert_idx, mol_pert_idx, mol_doses,
           routing_table, assay_table, cell_type_table, experiment_table,
           well_table, gene_table, mol_table,
           fourier_freqs_time, fourier_freqs_dose):
    time_emb = _fourier_pallas(time, fourier_freqs_time)
    dose_emb = _fourier_pallas(mol_doses, fourier_freqs_dose)

    out = _sc_assemble()(
        time_emb, xt, dose_emb,
        routing_idx,
        assay_idx,
        cell_type_idx,
        experiment_idx,
        well_idx,
        gene_pert_idx,
        mol_pert_idx,
        routing_table, assay_table, cell_type_table, experiment_table,
        well_table, gene_table, mol_table,
    )
    return out


# speedup vs baseline: 1.8866x; 1.8866x over previous
import functools
import jax
import jax.numpy as jnp
from jax import lax
from jax.experimental import pallas as pl
from jax.experimental.pallas import tpu as pltpu
from jax.experimental.pallas import tpu_sc as plsc

B = 16384
D = 64
NC, NS = 2, 16
NW = NC * NS
CHUNK = B // NW
_TWO_PI = 6.283185307179586
BLK = 512


def _sc_gather_body(g_idx, m_idx, g_tab, m_tab, rows, *scratch):
    idxb = scratch[0:6]
    bufs = scratch[6:8]
    gsems = scratch[8:10]
    ssems = scratch[10:12]
    isem = scratch[12]
    wid = lax.axis_index("s") * NC + lax.axis_index("c")
    base = wid * CHUNK
    jobs = [(j, g_idx if j < 3 else m_idx, (j % 3) * B,
             g_tab if j < 3 else m_tab) for j in range(6)]
    ids_ = []
    for j, (slot, idx_hbm, off, tab) in enumerate(jobs):
        ids_.append(pltpu.async_copy(idx_hbm.at[pl.ds(off + base, CHUNK)],
                                     idxb[j], isem))
    sds = [None, None]
    prev = None
    for j, (slot, idx_hbm, off, tab) in enumerate(jobs):
        b = j % 2
        if sds[b] is not None:
            sds[b].wait()
        ids_[j].wait()
        gd = pltpu.async_copy(tab.at[idxb[j]], bufs[b], gsems[b])
        if prev is not None:
            pgd, pslot, pb = prev
            pgd.wait()
            sds[pb] = pltpu.async_copy(
                bufs[pb], rows.at[pslot, pl.ds(base, CHUNK)], ssems[pb])
        prev = (gd, slot, b)
    pgd, pslot, pb = prev
    pgd.wait()
    sds[pb] = pltpu.async_copy(bufs[pb],
                               rows.at[pslot, pl.ds(base, CHUNK)],
                               ssems[pb])
    for sd in sds:
        if sd is not None:
            sd.wait()


@functools.cache
def _sc_gather():
    return pl.kernel(
        _sc_gather_body,
        out_type=jax.ShapeDtypeStruct((6, B, D), jnp.float32),
        mesh=plsc.VectorSubcoreMesh(core_axis_name="c", subcore_axis_name="s",
                                    num_cores=NC, num_subcores=NS),
        scratch_types=(
            [pltpu.VMEM((CHUNK,), jnp.int32) for _ in range(6)]
            + [pltpu.VMEM((CHUNK, D), jnp.float32) for _ in range(2)]
            + [pltpu.SemaphoreType.DMA] * 5
        ),
        compiler_params=pltpu.CompilerParams(use_tc_tiling_on_sc=False),
    )


_SMALLS = [(2, 4), (3, 16), (4, 256), (5, 1024), (6, 384)]


def _tc_assemble_body(time_ref, xt_t_ref, dose_ref,
                      ri_ref, ai_ref, ci_ref, ei_ref, wi_ref,
                      rt_ref, at_ref, ct_ref, et_ref, wt_ref,
                      ft_ref, fd_ref, rows_ref, o_ref):
    f_t = ft_ref[:]
    t = time_ref[:]
    ang = _TWO_PI * f_t * t[None, :]
    o_ref[0, : D // 2, :] = jnp.sin(ang)
    o_ref[0, D // 2 :, :] = jnp.cos(ang)
    o_ref[1] = xt_t_ref[:]
    idx_refs = [ri_ref, ai_ref, ci_ref, ei_ref, wi_ref]
    tab_refs = [rt_ref, at_ref, ct_ref, et_ref, wt_ref]
    for k, (slot, R) in enumerate(_SMALLS):
        idx = idx_refs[k][:]
        onehot = (idx[None, :] == lax.broadcasted_iota(jnp.int32, (R, BLK), 0)
                  ).astype(jnp.float32)
        o_ref[slot] = jnp.dot(tab_refs[k][:], onehot,
                              preferred_element_type=jnp.float32)
    for j in range(6):
        o_ref[7 + j] = rows_ref[j, :, :].T
    f_d = fd_ref[:]
    dv = dose_ref[:]
    for j in range(3):
        angd = _TWO_PI * f_d * dv[j][None, :]
        o_ref[13 + j, : D // 2, :] = jnp.sin(angd)
        o_ref[13 + j, D // 2 :, :] = jnp.cos(angd)


def _tc_assemble(time, xt_t, doses2, r_i, a_i, c_i, e_i, w_i,
                 rt, at, ct, et, wt, ft, fd, rows):
    grid = (B // BLK,)
    ispec = [
        pl.BlockSpec((BLK,), lambda i: (i,)),
        pl.BlockSpec((D, BLK), lambda i: (0, i)),
        pl.BlockSpec((3, BLK), lambda i: (0, i)),
    ] + [pl.BlockSpec((BLK,), lambda i: (i,)) for _ in range(5)] + [
        pl.BlockSpec((D, R), lambda i: (0, 0)) for _, R in _SMALLS
    ] + [
        pl.BlockSpec((D // 2, 1), lambda i: (0, 0)),
        pl.BlockSpec((D // 2, 1), lambda i: (0, 0)),
        pl.BlockSpec((6, BLK, D), lambda i: (0, i, 0)),
    ]
    return pl.pallas_call(
        _tc_assemble_body,
        grid=grid,
        in_specs=ispec,
        out_specs=pl.BlockSpec((16, D, BLK), lambda i: (0, 0, i)),
        out_shape=jax.ShapeDtypeStruct((16, D, B), jnp.float32),
    )(time, xt_t, doses2, r_i, a_i, c_i, e_i, w_i,
      rt, at, ct, et, wt, ft.reshape(D // 2, 1), fd.reshape(D // 2, 1), rows)


def kernel(time, xt, routing_idx, assay_idx, cell_type_idx, experiment_idx,
           well_idx, gene_pert_idx, mol_pert_idx, mol_doses,
           routing_table, assay_table, cell_type_table, experiment_table,
           well_table, gene_table, mol_table,
           fourier_freqs_time, fourier_freqs_dose):
    rows3 = _sc_gather()(gene_pert_idx, mol_pert_idx, gene_table, mol_table)
    out_t = _tc_assemble(time, xt.T, mol_doses.reshape(3, B),
                         routing_idx, assay_idx, cell_type_idx,
                         experiment_idx, well_idx,
                         routing_table.T, assay_table.T, cell_type_table.T,
                         experiment_table.T, well_table.T,
                         fourier_freqs_time, fourier_freqs_dose, rows3)
    return jnp.swapaxes(out_t, 1, 2)


# speedup vs baseline: 2.3228x; 1.2312x over previous
import functools
import jax
import jax.numpy as jnp
from jax import lax
from jax.experimental import pallas as pl
from jax.experimental.pallas import tpu as pltpu
from jax.experimental.pallas import tpu_sc as plsc

B = 16384
D = 64
NC, NS = 2, 16
NW = NC * NS
CHUNK = B // NW
_TWO_PI = 6.283185307179586
BLK = 512


DP = 2 * D
SUB = 256
NSUB = CHUNK // SUB


def _pad_body(t_ref, o_ref):
    o_ref[:, :D] = t_ref[:].T
    o_ref[:, D:] = jnp.zeros_like(o_ref[:, D:])


def _pad_table(table_t, blk):
    n = table_t.shape[1]
    return pl.pallas_call(
        _pad_body,
        grid=((n + blk - 1) // blk,),
        in_specs=[pl.BlockSpec((D, blk), lambda i: (0, i))],
        out_specs=pl.BlockSpec((blk, DP), lambda i: (i, 0)),
        out_shape=jax.ShapeDtypeStruct((n, DP), jnp.float32),
    )(table_t)


def _sc_gather_body(g_idx, m_idx, g_tab, m_tab, rows, *scratch):
    idxb = scratch[0:12]
    bufs = scratch[12:14]
    gsems = scratch[14:16]
    ssems = scratch[16:18]
    isem = scratch[18]
    wid = lax.axis_index("s") * NC + lax.axis_index("c")
    base = wid * CHUNK
    jobs = []
    for slot in range(6):
        idx_hbm = g_idx if slot < 3 else m_idx
        tab = g_tab if slot < 3 else m_tab
        for k in range(NSUB):
            jobs.append((slot, idx_hbm, (slot % 3) * B + k * SUB, tab,
                         k * SUB))
    ids_ = []
    for j, (slot, idx_hbm, off, tab, sub) in enumerate(jobs):
        ids_.append(pltpu.async_copy(idx_hbm.at[pl.ds(off + base, SUB)],
                                     idxb[j], isem))
    sds = [None, None]
    prev = None
    for j, (slot, idx_hbm, off, tab, sub) in enumerate(jobs):
        b = j % 2
        if sds[b] is not None:
            sds[b].wait()
        ids_[j].wait()
        gd = pltpu.async_copy(tab.at[idxb[j]], bufs[b], gsems[b])
        if prev is not None:
            pgd, pslot, psub, pb = prev
            pgd.wait()
            sds[pb] = pltpu.async_copy(
                bufs[pb], rows.at[pslot, pl.ds(base + psub, SUB)], ssems[pb])
        prev = (gd, slot, sub, b)
    pgd, pslot, psub, pb = prev
    pgd.wait()
    sds[pb] = pltpu.async_copy(bufs[pb],
                               rows.at[pslot, pl.ds(base + psub, SUB)],
                               ssems[pb])
    for sd in sds:
        if sd is not None:
            sd.wait()


@functools.cache
def _sc_gather():
    return pl.kernel(
        _sc_gather_body,
        out_type=jax.ShapeDtypeStruct((6, B, DP), jnp.float32),
        mesh=plsc.VectorSubcoreMesh(core_axis_name="c", subcore_axis_name="s",
                                    num_cores=NC, num_subcores=NS),
        scratch_types=(
            [pltpu.VMEM((SUB,), jnp.int32) for _ in range(12)]
            + [pltpu.VMEM((SUB, DP), jnp.float32) for _ in range(2)]
            + [pltpu.SemaphoreType.DMA] * 5
        ),
        compiler_params=pltpu.CompilerParams(use_tc_tiling_on_sc=True),
    )


_SMALLS = [(2, 4), (3, 16), (4, 256), (5, 1024), (6, 384)]


def _tc_assemble_body(time_ref, xt_t_ref, dose_ref,
                      ri_ref, ai_ref, ci_ref, ei_ref, wi_ref,
                      rt_ref, at_ref, ct_ref, et_ref, wt_ref,
                      ft_ref, fd_ref, rows_ref, o_ref):
    f_t = ft_ref[:]
    t = time_ref[:]
    ang = _TWO_PI * f_t * t[None, :]
    o_ref[0, : D // 2, :] = jnp.sin(ang)
    o_ref[0, D // 2 :, :] = jnp.cos(ang)
    o_ref[1] = xt_t_ref[:]
    idx_refs = [ri_ref, ai_ref, ci_ref, ei_ref, wi_ref]
    tab_refs = [rt_ref, at_ref, ct_ref, et_ref, wt_ref]
    for k, (slot, R) in enumerate(_SMALLS):
        idx = idx_refs[k][:]
        onehot = (idx[None, :] == lax.broadcasted_iota(jnp.int32, (R, BLK), 0)
                  ).astype(jnp.float32)
        o_ref[slot] = jnp.dot(tab_refs[k][:], onehot,
                              preferred_element_type=jnp.float32)
    for j in range(6):
        o_ref[7 + j] = rows_ref[j, :, :D].T
    f_d = fd_ref[:]
    dv = dose_ref[:]
    for j in range(3):
        angd = _TWO_PI * f_d * dv[j][None, :]
        o_ref[13 + j, : D // 2, :] = jnp.sin(angd)
        o_ref[13 + j, D // 2 :, :] = jnp.cos(angd)


def _tc_assemble(time, xt_t, doses2, r_i, a_i, c_i, e_i, w_i,
                 rt, at, ct, et, wt, ft, fd, rows):
    grid = (B // BLK,)
    ispec = [
        pl.BlockSpec((BLK,), lambda i: (i,)),
        pl.BlockSpec((D, BLK), lambda i: (0, i)),
        pl.BlockSpec((3, BLK), lambda i: (0, i)),
    ] + [pl.BlockSpec((BLK,), lambda i: (i,)) for _ in range(5)] + [
        pl.BlockSpec((D, R), lambda i: (0, 0)) for _, R in _SMALLS
    ] + [
        pl.BlockSpec((D // 2, 1), lambda i: (0, 0)),
        pl.BlockSpec((D // 2, 1), lambda i: (0, 0)),
        pl.BlockSpec((6, BLK, DP), lambda i: (0, i, 0)),
    ]
    return pl.pallas_call(
        _tc_assemble_body,
        grid=grid,
        in_specs=ispec,
        out_specs=pl.BlockSpec((16, D, BLK), lambda i: (0, 0, i)),
        out_shape=jax.ShapeDtypeStruct((16, D, B), jnp.float32),
    )(time, xt_t, doses2, r_i, a_i, c_i, e_i, w_i,
      rt, at, ct, et, wt, ft.reshape(D // 2, 1), fd.reshape(D // 2, 1), rows)


def kernel(time, xt, routing_idx, assay_idx, cell_type_idx, experiment_idx,
           well_idx, gene_pert_idx, mol_pert_idx, mol_doses,
           routing_table, assay_table, cell_type_table, experiment_table,
           well_table, gene_table, mol_table,
           fourier_freqs_time, fourier_freqs_dose):
    gene128 = _pad_table(gene_table.T, 2048)
    mol128 = _pad_table(mol_table.T, 2048)
    rows3 = _sc_gather()(gene_pert_idx, mol_pert_idx, gene128, mol128)
    out_t = _tc_assemble(time, xt.T, mol_doses.reshape(3, B),
                         routing_idx, assay_idx, cell_type_idx,
                         experiment_idx, well_idx,
                         routing_table.T, assay_table.T, cell_type_table.T,
                         experiment_table.T, well_table.T,
                         fourier_freqs_time, fourier_freqs_dose, rows3)
    return jnp.swapaxes(out_t, 1, 2)


# speedup vs baseline: 2.3258x; 1.0013x over previous
import functools
import jax
import jax.numpy as jnp
from jax import lax
from jax.experimental import pallas as pl
from jax.experimental.pallas import tpu as pltpu
from jax.experimental.pallas import tpu_sc as plsc

B = 16384
D = 64
NC, NS = 2, 16
NW = NC * NS
CHUNK = B // NW
_TWO_PI = 6.283185307179586
BLK = 512


DP = 2 * D
SUB = 256
NSUB = CHUNK // SUB


def _pad_body(t_ref, o_ref):
    o_ref[:, :D] = t_ref[:].T
    o_ref[:, D:] = jnp.zeros_like(o_ref[:, D:])


def _pad_table(table_t, blk):
    n = table_t.shape[1]
    return pl.pallas_call(
        _pad_body,
        grid=((n + blk - 1) // blk,),
        in_specs=[pl.BlockSpec((D, blk), lambda i: (0, i))],
        out_specs=pl.BlockSpec((blk, DP), lambda i: (i, 0)),
        out_shape=jax.ShapeDtypeStruct((n, DP), jnp.float32),
    )(table_t)


def _sc_gather_body(g_idx, m_idx, g_tab, m_tab, rows, *scratch):
    idxb = scratch[0:2]
    bufs = scratch[2:4]
    gsems = scratch[4:6]
    ssems = scratch[6:8]
    wid = lax.axis_index("s") * NC + lax.axis_index("c")
    base = wid * CHUNK
    jobs = []
    for slot in range(6):
        idx_hbm = g_idx if slot < 3 else m_idx
        tab = g_tab if slot < 3 else m_tab
        for k in range(NSUB):
            jobs.append((slot, idx_hbm, (slot % 3) * B + k * SUB, tab,
                         k * SUB))
    sds = [None, None]
    prev = None
    for j, (slot, idx_hbm, off, tab, sub) in enumerate(jobs):
        b = j % 2
        if sds[b] is not None:
            sds[b].wait()
        pltpu.sync_copy(idx_hbm.at[pl.ds(off + base, SUB)], idxb[b])
        gd = pltpu.async_copy(tab.at[idxb[b]], bufs[b], gsems[b])
        if prev is not None:
            pgd, pslot, psub, pb = prev
            pgd.wait()
            sds[pb] = pltpu.async_copy(
                bufs[pb], rows.at[pslot, pl.ds(base + psub, SUB)], ssems[pb])
        prev = (gd, slot, sub, b)
    pgd, pslot, psub, pb = prev
    pgd.wait()
    sds[pb] = pltpu.async_copy(bufs[pb],
                               rows.at[pslot, pl.ds(base + psub, SUB)],
                               ssems[pb])
    for sd in sds:
        if sd is not None:
            sd.wait()


@functools.cache
def _sc_gather():
    return pl.kernel(
        _sc_gather_body,
        out_type=jax.ShapeDtypeStruct((6, B, DP), jnp.float32),
        mesh=plsc.VectorSubcoreMesh(core_axis_name="c", subcore_axis_name="s",
                                    num_cores=NC, num_subcores=NS),
        scratch_types=(
            [pltpu.VMEM((SUB,), jnp.int32) for _ in range(2)]
            + [pltpu.VMEM((SUB, DP), jnp.float32) for _ in range(2)]
            + [pltpu.SemaphoreType.DMA] * 4
        ),
        compiler_params=pltpu.CompilerParams(use_tc_tiling_on_sc=True),
    )


_SMALLS = [(2, 4), (3, 16), (4, 256), (5, 1024), (6, 384)]


def _tc_assemble_body(time_ref, xt_t_ref, dose_ref,
                      ri_ref, ai_ref, ci_ref, ei_ref, wi_ref,
                      rt_ref, at_ref, ct_ref, et_ref, wt_ref,
                      ft_ref, fd_ref, rows_ref, o_ref):
    f_t = ft_ref[:]
    t = time_ref[:]
    ang = _TWO_PI * f_t * t[None, :]
    o_ref[0, : D // 2, :] = jnp.sin(ang)
    o_ref[0, D // 2 :, :] = jnp.cos(ang)
    o_ref[1] = xt_t_ref[:]
    idx_refs = [ri_ref, ai_ref, ci_ref, ei_ref, wi_ref]
    tab_refs = [rt_ref, at_ref, ct_ref, et_ref, wt_ref]
    for k, (slot, R) in enumerate(_SMALLS):
        idx = idx_refs[k][:]
        onehot = (idx[None, :] == lax.broadcasted_iota(jnp.int32, (R, BLK), 0)
                  ).astype(jnp.float32)
        o_ref[slot] = jnp.dot(tab_refs[k][:], onehot,
                              preferred_element_type=jnp.float32)
    for j in range(6):
        o_ref[7 + j] = rows_ref[j, :, :D].T
    f_d = fd_ref[:]
    dv = dose_ref[:]
    for j in range(3):
        angd = _TWO_PI * f_d * dv[j][None, :]
        o_ref[13 + j, : D // 2, :] = jnp.sin(angd)
        o_ref[13 + j, D // 2 :, :] = jnp.cos(angd)


def _tc_assemble(time, xt_t, doses2, r_i, a_i, c_i, e_i, w_i,
                 rt, at, ct, et, wt, ft, fd, rows):
    grid = (B // BLK,)
    ispec = [
        pl.BlockSpec((BLK,), lambda i: (i,)),
        pl.BlockSpec((D, BLK), lambda i: (0, i)),
        pl.BlockSpec((3, BLK), lambda i: (0, i)),
    ] + [pl.BlockSpec((BLK,), lambda i: (i,)) for _ in range(5)] + [
        pl.BlockSpec((D, R), lambda i: (0, 0)) for _, R in _SMALLS
    ] + [
        pl.BlockSpec((D // 2, 1), lambda i: (0, 0)),
        pl.BlockSpec((D // 2, 1), lambda i: (0, 0)),
        pl.BlockSpec((6, BLK, DP), lambda i: (0, i, 0)),
    ]
    return pl.pallas_call(
        _tc_assemble_body,
        grid=grid,
        in_specs=ispec,
        out_specs=pl.BlockSpec((16, D, BLK), lambda i: (0, 0, i)),
        out_shape=jax.ShapeDtypeStruct((16, D, B), jnp.float32),
    )(time, xt_t, doses2, r_i, a_i, c_i, e_i, w_i,
      rt, at, ct, et, wt, ft.reshape(D // 2, 1), fd.reshape(D // 2, 1), rows)


def kernel(time, xt, routing_idx, assay_idx, cell_type_idx, experiment_idx,
           well_idx, gene_pert_idx, mol_pert_idx, mol_doses,
           routing_table, assay_table, cell_type_table, experiment_table,
           well_table, gene_table, mol_table,
           fourier_freqs_time, fourier_freqs_dose):
    gene128 = _pad_table(gene_table.T, 2048)
    mol128 = _pad_table(mol_table.T, 2048)
    rows3 = _sc_gather()(gene_pert_idx, mol_pert_idx, gene128, mol128)
    out_t = _tc_assemble(time, xt.T, mol_doses.reshape(3, B),
                         routing_idx, assay_idx, cell_type_idx,
                         experiment_idx, well_idx,
                         routing_table.T, assay_table.T, cell_type_table.T,
                         experiment_table.T, well_table.T,
                         fourier_freqs_time, fourier_freqs_dose, rows3)
    return jnp.swapaxes(out_t, 1, 2)
